# baseline (speedup 1.0000x reference)
import jax
import jax.numpy as jnp
from jax.experimental import pallas as pl
from jax.experimental.pallas import tpu as pltpu

_N = 10000
_ROWS = 200


def _gcn_body(feat_ref, w_ref, adj_ref, out_ref, support_ref):
    @pl.when(pl.program_id(0) == 0)
    def _compute_support():
        sup = jax.lax.dot_general(
            feat_ref[...], w_ref[...],
            dimension_numbers=(((1,), (0,)), ((), ())),
            precision=jax.lax.Precision.HIGHEST,
            preferred_element_type=jnp.float32,
        )
        support_ref[...] = sup.astype(jnp.bfloat16)

    acc = jax.lax.dot_general(
        adj_ref[...].astype(jnp.bfloat16), support_ref[...],
        dimension_numbers=(((1,), (0,)), ((), ())),
        preferred_element_type=jnp.float32,
    )
    out_ref[...] = jnp.maximum(acc, 0.0)


def kernel(features, adj, weight):
    n, in_f = features.shape
    out_f = weight.shape[1]
    grid = (n // _ROWS,) if n % _ROWS == 0 else (pl.cdiv(n, _ROWS),)
    return pl.pallas_call(
        _gcn_body,
        grid=grid,
        in_specs=[
            pl.BlockSpec((n, in_f), lambda i: (0, 0)),
            pl.BlockSpec((in_f, out_f), lambda i: (0, 0)),
            pl.BlockSpec((_ROWS, n), lambda i: (i, 0)),
        ],
        out_specs=pl.BlockSpec((_ROWS, out_f), lambda i: (i, 0)),
        out_shape=jax.ShapeDtypeStruct((n, out_f), jnp.float32),
        scratch_shapes=[pltpu.VMEM((n, out_f), jnp.bfloat16)],
    )(features, weight, adj)
